# baseline (speedup 1.0000x reference)
import jax
import jax.numpy as jnp
from jax.experimental import pallas as pl
from jax.experimental.pallas import tpu as pltpu

B = 4096
DIM = 1024
NUM_EXPERTS = 8
NUM_HEADS = 16
TOP_K = 2
HEAD_DIM = DIM // NUM_HEADS
SCALE = HEAD_DIM ** (-0.5)

BT_ATTN = 512
CHUNK = 512
NCHUNK = 3 * DIM // CHUNK
Q_CHUNKS = DIM // CHUNK


def _routing_weights(scores):
    bt = scores.shape[0]
    e_iota = jax.lax.broadcasted_iota(jnp.int32, (bt, NUM_EXPERTS), 1)
    m1 = jnp.max(scores, axis=1, keepdims=True)
    idx1 = jnp.min(jnp.where(scores == m1, e_iota, NUM_EXPERTS), axis=1,
                   keepdims=True)
    masked = jnp.where(e_iota == idx1, -1.0, scores)
    m2 = jnp.max(masked, axis=1, keepdims=True)
    idx2 = jnp.min(jnp.where(masked == m2, e_iota, NUM_EXPERTS), axis=1,
                   keepdims=True)
    return jnp.where(e_iota == idx1, m1, 0.0) + jnp.where(e_iota == idx2, m2, 0.0)


def _router_kernel(x_ref, wg_ref, bg_ref, gates_ref):
    scores = jnp.dot(x_ref[...], wg_ref[...],
                     preferred_element_type=jnp.float32) + bg_ref[...]
    scores = scores - jnp.max(scores, axis=1, keepdims=True)
    scores = jnp.exp(scores)
    scores = scores / jnp.sum(scores, axis=1, keepdims=True)
    gates_ref[...] = _routing_weights(scores)


def _expert_kernel(xb_ref, yb_ref, w_ref, gates_ref, qkv_ref):
    c = pl.program_id(0)
    e = pl.program_id(1)
    wb = w_ref[0].astype(jnp.bfloat16)
    gates = gates_ref[...]
    lane = jax.lax.broadcasted_iota(jnp.int32, gates.shape, 1)
    we = jnp.sum(jnp.where(lane == e, gates, 0.0), axis=1, keepdims=True)

    def accumulate(src):
        contrib = we * jnp.dot(src, wb, preferred_element_type=jnp.float32)

        @pl.when(e == 0)
        def _():
            qkv_ref[...] = contrib

        @pl.when(e > 0)
        def _():
            qkv_ref[...] += contrib

    @pl.when(c < Q_CHUNKS)
    def _():
        accumulate(yb_ref[...])

    @pl.when(c >= Q_CHUNKS)
    def _():
        accumulate(xb_ref[...])


def _attn_kernel(qkv_ref, wp_ref, bp_ref, out_ref):
    bt = qkv_ref.shape[0]
    q3 = qkv_ref[:, :DIM].reshape(bt, NUM_HEADS, HEAD_DIM)
    k3 = qkv_ref[:, DIM:2 * DIM].reshape(bt, NUM_HEADS, HEAD_DIM)
    v3 = qkv_ref[:, 2 * DIM:].reshape(bt, NUM_HEADS, HEAD_DIM)
    attn = jax.lax.dot_general(
        q3, k3, (((2,), (2,)), ((0,), (0,))),
        preferred_element_type=jnp.float32) * SCALE
    attn = attn - jnp.max(attn, axis=2, keepdims=True)
    attn = jnp.exp(attn)
    attn = attn / jnp.sum(attn, axis=2, keepdims=True)
    ctx = jax.lax.dot_general(
        attn, v3, (((2,), (1,)), ((0,), (0,))),
        preferred_element_type=jnp.float32)
    ctx = ctx.reshape(bt, DIM)
    out_ref[...] = jnp.dot(ctx, wp_ref[...],
                           preferred_element_type=jnp.float32) + bp_ref[...]


@jax.jit
def kernel(x, y, W_qkv, W_gate, b_gate, W_proj, b_proj):
    xb = x.astype(jnp.bfloat16)
    yb = y.astype(jnp.bfloat16)
    W_proj_perm = (W_proj.reshape(HEAD_DIM, NUM_HEADS, DIM)
                   .transpose(1, 0, 2).reshape(DIM, DIM))

    gates = pl.pallas_call(
        _router_kernel,
        grid=(1,),
        in_specs=[
            pl.BlockSpec((B, DIM), lambda i: (0, 0)),
            pl.BlockSpec((DIM, NUM_EXPERTS), lambda i: (0, 0)),
            pl.BlockSpec((1, NUM_EXPERTS), lambda i: (0, 0)),
        ],
        out_specs=pl.BlockSpec((B, NUM_EXPERTS), lambda i: (0, 0)),
        out_shape=jax.ShapeDtypeStruct((B, NUM_EXPERTS), jnp.float32),
    )(x, W_gate, b_gate.reshape(1, NUM_EXPERTS))

    qkv = pl.pallas_call(
        _expert_kernel,
        grid=(NCHUNK, NUM_EXPERTS),
        in_specs=[
            pl.BlockSpec((B, DIM), lambda c, e: (0, 0)),
            pl.BlockSpec((B, DIM), lambda c, e: (0, 0)),
            pl.BlockSpec((1, DIM, CHUNK), lambda c, e: (e, 0, c)),
            pl.BlockSpec((B, NUM_EXPERTS), lambda c, e: (0, 0)),
        ],
        out_specs=pl.BlockSpec((B, CHUNK), lambda c, e: (0, c)),
        out_shape=jax.ShapeDtypeStruct((B, 3 * DIM), jnp.float32),
        compiler_params=pltpu.CompilerParams(
            dimension_semantics=("arbitrary", "arbitrary"),
        ),
    )(xb, yb, W_qkv, gates)

    out = pl.pallas_call(
        _attn_kernel,
        grid=(B // BT_ATTN,),
        in_specs=[
            pl.BlockSpec((BT_ATTN, 3 * DIM), lambda t: (t, 0)),
            pl.BlockSpec((DIM, DIM), lambda t: (0, 0)),
            pl.BlockSpec((1, DIM), lambda t: (0, 0)),
        ],
        out_specs=pl.BlockSpec((BT_ATTN, DIM), lambda t: (t, 0)),
        out_shape=jax.ShapeDtypeStruct((B, DIM), jnp.float32),
        compiler_params=pltpu.CompilerParams(
            dimension_semantics=("arbitrary",),
        ),
    )(qkv, W_proj_perm, b_proj.reshape(1, DIM))
    return out


# speedup vs baseline: 1.0515x; 1.0515x over previous
import jax
import jax.numpy as jnp
from jax.experimental import pallas as pl
from jax.experimental.pallas import tpu as pltpu

B = 4096
DIM = 1024
NUM_EXPERTS = 8
NUM_HEADS = 16
TOP_K = 2
HEAD_DIM = DIM // NUM_HEADS
SCALE = HEAD_DIM ** (-0.5)

BT_ATTN = 512
CHUNK = 512
NCHUNK = 3 * DIM // CHUNK
Q_CHUNKS = DIM // CHUNK


def _routing_weights(scores):
    bt = scores.shape[0]
    e_iota = jax.lax.broadcasted_iota(jnp.int32, (bt, NUM_EXPERTS), 1)
    m1 = jnp.max(scores, axis=1, keepdims=True)
    idx1 = jnp.min(jnp.where(scores == m1, e_iota, NUM_EXPERTS), axis=1,
                   keepdims=True)
    masked = jnp.where(e_iota == idx1, -1.0, scores)
    m2 = jnp.max(masked, axis=1, keepdims=True)
    idx2 = jnp.min(jnp.where(masked == m2, e_iota, NUM_EXPERTS), axis=1,
                   keepdims=True)
    return jnp.where(e_iota == idx1, m1, 0.0) + jnp.where(e_iota == idx2, m2, 0.0)


def _router_kernel(x_ref, wg_ref, bg_ref, gates_ref):
    scores = jnp.dot(x_ref[...], wg_ref[...],
                     preferred_element_type=jnp.float32) + bg_ref[...]
    scores = scores - jnp.max(scores, axis=1, keepdims=True)
    scores = jnp.exp(scores)
    scores = scores / jnp.sum(scores, axis=1, keepdims=True)
    gates_ref[...] = _routing_weights(scores)


def _expert_kernel(xb_ref, yb_ref, w_ref, gates_ref, qkv_ref):
    c = pl.program_id(0)
    e = pl.program_id(1)
    wb = w_ref[0].astype(jnp.bfloat16)
    gates = gates_ref[...]
    lane = jax.lax.broadcasted_iota(jnp.int32, gates.shape, 1)
    we = jnp.sum(jnp.where(lane == e, gates, 0.0), axis=1, keepdims=True)

    def accumulate(src):
        contrib = (we * jnp.dot(src, wb, preferred_element_type=jnp.float32)
                   ).astype(jnp.bfloat16)

        @pl.when(e == 0)
        def _():
            qkv_ref[...] = contrib

        @pl.when(e > 0)
        def _():
            qkv_ref[...] += contrib

    @pl.when(c < Q_CHUNKS)
    def _():
        accumulate(yb_ref[...])

    @pl.when(c >= Q_CHUNKS)
    def _():
        accumulate(xb_ref[...])


def _attn_kernel(qkv_ref, wp_ref, bp_ref, out_ref):
    bt = qkv_ref.shape[0]
    q3 = qkv_ref[:, :DIM].reshape(bt, NUM_HEADS, HEAD_DIM)
    k3 = qkv_ref[:, DIM:2 * DIM].reshape(bt, NUM_HEADS, HEAD_DIM)
    v3 = qkv_ref[:, 2 * DIM:].reshape(bt, NUM_HEADS, HEAD_DIM)
    attn = jax.lax.dot_general(
        q3, k3, (((2,), (2,)), ((0,), (0,))),
        preferred_element_type=jnp.float32) * SCALE
    attn = attn - jnp.max(attn, axis=2, keepdims=True)
    attn = jnp.exp(attn)
    attn = (attn / jnp.sum(attn, axis=2, keepdims=True)).astype(v3.dtype)
    ctx = jax.lax.dot_general(
        attn, v3, (((2,), (1,)), ((0,), (0,))),
        preferred_element_type=jnp.float32)
    ctx = ctx.reshape(bt, DIM)
    out_ref[...] = jnp.dot(ctx, wp_ref[...],
                           preferred_element_type=jnp.float32) + bp_ref[...]


@jax.jit
def kernel(x, y, W_qkv, W_gate, b_gate, W_proj, b_proj):
    xb = x.astype(jnp.bfloat16)
    yb = y.astype(jnp.bfloat16)
    W_proj_perm = (W_proj.reshape(HEAD_DIM, NUM_HEADS, DIM)
                   .transpose(1, 0, 2).reshape(DIM, DIM))

    gates = pl.pallas_call(
        _router_kernel,
        grid=(1,),
        in_specs=[
            pl.BlockSpec((B, DIM), lambda i: (0, 0)),
            pl.BlockSpec((DIM, NUM_EXPERTS), lambda i: (0, 0)),
            pl.BlockSpec((1, NUM_EXPERTS), lambda i: (0, 0)),
        ],
        out_specs=pl.BlockSpec((B, NUM_EXPERTS), lambda i: (0, 0)),
        out_shape=jax.ShapeDtypeStruct((B, NUM_EXPERTS), jnp.float32),
    )(x, W_gate, b_gate.reshape(1, NUM_EXPERTS))

    qkv = pl.pallas_call(
        _expert_kernel,
        grid=(NCHUNK, NUM_EXPERTS),
        in_specs=[
            pl.BlockSpec((B, DIM), lambda c, e: (0, 0)),
            pl.BlockSpec((B, DIM), lambda c, e: (0, 0)),
            pl.BlockSpec((1, DIM, CHUNK), lambda c, e: (e, 0, c)),
            pl.BlockSpec((B, NUM_EXPERTS), lambda c, e: (0, 0)),
        ],
        out_specs=pl.BlockSpec((B, CHUNK), lambda c, e: (0, c)),
        out_shape=jax.ShapeDtypeStruct((B, 3 * DIM), jnp.bfloat16),
        compiler_params=pltpu.CompilerParams(
            dimension_semantics=("arbitrary", "arbitrary"),
        ),
    )(xb, yb, W_qkv, gates)

    out = pl.pallas_call(
        _attn_kernel,
        grid=(B // BT_ATTN,),
        in_specs=[
            pl.BlockSpec((BT_ATTN, 3 * DIM), lambda t: (t, 0)),
            pl.BlockSpec((DIM, DIM), lambda t: (0, 0)),
            pl.BlockSpec((1, DIM), lambda t: (0, 0)),
        ],
        out_specs=pl.BlockSpec((BT_ATTN, DIM), lambda t: (t, 0)),
        out_shape=jax.ShapeDtypeStruct((B, DIM), jnp.float32),
        compiler_params=pltpu.CompilerParams(
            dimension_semantics=("arbitrary",),
        ),
    )(qkv, W_proj_perm, b_proj.reshape(1, DIM))
    return out
